# Initial kernel scaffold; baseline (speedup 1.0000x reference)
import jax
import jax.numpy as jnp
from jax.experimental import pallas as pl


def kernel(text_ids, emb_table):
    raise NotImplementedError("write your pallas kernel here")


# speedup vs baseline: 1.1457x; 1.1457x over previous
import functools
import math

import jax
import jax.numpy as jnp
import numpy as np
from jax import lax
from jax.experimental import pallas as pl
from jax.experimental.pallas import tpu as pltpu
from jax.experimental.pallas import tpu_sc as plsc

VOCAB = 100000
D_MODEL = 128
BATCH = 1024
SEQ = 50

_INFO = plsc.get_sparse_core_info()
_NC, _NS, _L = _INFO.num_cores, _INFO.num_subcores, _INFO.num_lanes
_NW = _NC * _NS
_ROWS = BATCH * SEQ
_ROWS_PER_W = _ROWS // _NW
_B_PER_CHUNK = 8
_CHUNK = _B_PER_CHUNK * SEQ
_NCHUNK = _ROWS_PER_W // _CHUNK
_LANE_CHUNKS = D_MODEL // _L
_SCALE = math.sqrt(float(D_MODEL))


def _pos_enc(seq_len, d_model):
    pos = np.arange(seq_len)[:, None].astype(np.float32)
    div = np.exp(
        np.arange(0, d_model, 2).astype(np.float32) * -(np.log(10000.0) / d_model)
    )
    pe = np.zeros((seq_len, d_model), dtype=np.float32)
    pe[:, 0::2] = np.sin(pos * div)
    pe[:, 1::2] = np.cos(pos * div)
    return pe


_MESH = plsc.VectorSubcoreMesh(core_axis_name="c", subcore_axis_name="s")


@functools.partial(
    pl.kernel,
    mesh=_MESH,
    out_type=jax.ShapeDtypeStruct((_ROWS, D_MODEL), jnp.float32),
    scratch_types=[
        pltpu.VMEM((_CHUNK,), jnp.int32),
        pltpu.VMEM((_CHUNK, D_MODEL), jnp.float32),
        pltpu.VMEM((SEQ, D_MODEL), jnp.float32),
        pltpu.SemaphoreType.DMA,
    ],
)
def _emb_kernel(idx_hbm, table_hbm, pe_hbm, out_hbm, idx_v, rows_v, pe_v, sem):
    wid = lax.axis_index("s") * _NC + lax.axis_index("c")
    base = wid * _ROWS_PER_W
    pltpu.sync_copy(pe_hbm, pe_v)

    for ch in range(_NCHUNK):
        gbase = base + ch * _CHUNK
        pltpu.sync_copy(idx_hbm.at[pl.ds(gbase, _CHUNK)], idx_v)
        pltpu.async_copy(table_hbm.at[idx_v], rows_v, sem).wait()

        def body(s, carry):
            pes = [pe_v[s, pl.ds(c * _L, _L)] for c in range(_LANE_CHUNKS)]
            for bb in range(_B_PER_CHUNK):
                r = bb * SEQ + s
                for c in range(_LANE_CHUNKS):
                    sl = pl.ds(c * _L, _L)
                    rows_v[r, sl] = rows_v[r, sl] * _SCALE + pes[c]
            return carry

        lax.fori_loop(0, SEQ, body, 0)
        pltpu.sync_copy(rows_v, out_hbm.at[pl.ds(gbase, _CHUNK)])


def kernel(text_ids, emb_table):
    pe = jnp.asarray(_pos_enc(SEQ, D_MODEL))
    flat_ids = text_ids.reshape(_ROWS).astype(jnp.int32)
    out = _emb_kernel(flat_ids, emb_table, pe)
    return out.reshape(BATCH, SEQ, D_MODEL)
